# Initial kernel scaffold; baseline (speedup 1.0000x reference)
import jax
import jax.numpy as jnp
from jax.experimental import pallas as pl


def kernel(x, edge_index, W0, b0, Wl, bl, Wr, br, att, gb, W1, b1, W2, b2, W3, b3):
    raise NotImplementedError("write your pallas kernel here")


# speedup vs baseline: 2.0864x; 2.0864x over previous
import functools

import jax
import jax.numpy as jnp
from jax.experimental import pallas as pl
from jax.experimental.pallas import tpu as pltpu


def _gat_kernel(x_ref, src_ref, dst_ref, w0t_ref, b0_ref, wlt_ref, bl_ref,
                wrt_ref, br_ref, att_ref, gb_ref, o_ref, *, n_edges):
    f32 = jnp.float32
    x = x_ref[...]
    y0 = jnp.maximum(jnp.dot(x, w0t_ref[...], preferred_element_type=f32)
                     + b0_ref[...], 0.0)
    xl = jnp.dot(y0, wlt_ref[...], preferred_element_type=f32) + bl_ref[...]
    xr = jnp.dot(y0, wrt_ref[...], preferred_element_type=f32) + br_ref[...]

    n = x.shape[0]
    srcm = src_ref[...]
    dstm = dst_ref[...]
    rows, cols = srcm.shape
    ep = rows * cols
    ion = jax.lax.broadcasted_iota(jnp.int32, (rows, cols, n), 2)
    ohs = (srcm[:, :, None] == ion).astype(f32).reshape(ep, n)
    ohd = (dstm[:, :, None] == ion).astype(f32).reshape(ep, n)

    ide = jax.lax.broadcasted_iota(jnp.int32, (ep, 1), 0)
    maskf = (ide < n_edges).astype(f32)

    xls = jnp.dot(ohs, xl, preferred_element_type=f32)
    xrd = jnp.dot(ohd, xr, preferred_element_type=f32)
    e = xls + xrd
    e = jnp.where(e > 0, e, 0.2 * e)
    lg = jnp.dot(e, att_ref[...], preferred_element_type=f32)

    ohdm = ohd * maskf
    lgm = ohdm * lg + (ohdm - 1.0) * 1e30
    m = jnp.max(lgm, axis=0, keepdims=True)
    me = jnp.sum(ohd * m, axis=1, keepdims=True)
    ex = jnp.exp(lg - me) * maskf
    den = jnp.sum(ohd * ex, axis=0, keepdims=True)
    dene = jnp.sum(ohd * den, axis=1, keepdims=True)
    alpha = ex / (dene + 1e-16)

    msg = alpha * xls
    outg = jax.lax.dot_general(ohd, msg, (((0,), (0,)), ((), ())),
                               preferred_element_type=f32)
    o_ref[...] = outg + gb_ref[...]


def _dense_kernel(y_ref, w1_ref, b1_ref, w2_ref, b2_ref, w3_ref, b3_ref,
                  o_ref, acc_ref):
    i = pl.program_id(0)
    f32 = jnp.float32
    h = jnp.dot(w1_ref[...], y_ref[...], preferred_element_type=f32)
    h = jnp.maximum(h + b1_ref[...], 0.0)
    part = jnp.dot(w2_ref[...], h, preferred_element_type=f32)

    @pl.when(i == 0)
    def _():
        acc_ref[...] = part

    @pl.when(i > 0)
    def _():
        acc_ref[...] = acc_ref[...] + part

    @pl.when(i == pl.num_programs(0) - 1)
    def _():
        o2 = jnp.maximum(acc_ref[...] + b2_ref[...], 0.0)
        o_ref[...] = (jnp.dot(w3_ref[...], o2, preferred_element_type=f32)
                      + b3_ref[...])


def kernel(x, edge_index, W0, b0, Wl, bl, Wr, br, att, gb, W1, b1, W2, b2,
           W3, b3):
    f32 = jnp.float32
    n, in_c = x.shape
    h = W0.shape[0]
    e = edge_index.shape[1]
    out_c = W3.shape[0]
    n_edges = e + n
    ep = -(-n_edges // 1024) * 1024
    pad = ep - n_edges

    loop = jnp.arange(n, dtype=edge_index.dtype)
    zpad = jnp.zeros((pad,), dtype=edge_index.dtype)
    srcm = jnp.concatenate([edge_index[0], loop, zpad]).reshape(ep // 128, 128)
    dstm = jnp.concatenate([edge_index[1], loop, zpad]).reshape(ep // 128, 128)

    y = pl.pallas_call(
        functools.partial(_gat_kernel, n_edges=n_edges),
        out_shape=jax.ShapeDtypeStruct((n, h), f32),
    )(x, srcm, dstm,
      W0.T, b0.reshape(1, h),
      Wl.T, bl.reshape(1, h),
      Wr.T, br.reshape(1, h),
      att.reshape(h, 1), gb.reshape(1, h))

    nh = n * h
    yt = y.reshape(nh, 1)
    bm = 512
    grid = nh // bm

    out_pad = -(-out_c // 8) * 8
    w3p = jnp.pad(W3, ((0, out_pad - out_c), (0, 0)))
    b3p = jnp.pad(b3, (0, out_pad - out_c)).reshape(out_pad, 1)

    out = pl.pallas_call(
        _dense_kernel,
        grid=(grid,),
        in_specs=[
            pl.BlockSpec((nh, 1), lambda i: (0, 0)),
            pl.BlockSpec((bm, nh), lambda i: (i, 0)),
            pl.BlockSpec((bm, 1), lambda i: (i, 0)),
            pl.BlockSpec((h, bm), lambda i: (0, i)),
            pl.BlockSpec((h, 1), lambda i: (0, 0)),
            pl.BlockSpec((out_pad, h), lambda i: (0, 0)),
            pl.BlockSpec((out_pad, 1), lambda i: (0, 0)),
        ],
        out_specs=pl.BlockSpec((out_pad, 1), lambda i: (0, 0)),
        out_shape=jax.ShapeDtypeStruct((out_pad, 1), f32),
        scratch_shapes=[pltpu.VMEM((h, 1), f32)],
    )(yt, W1, b1.reshape(nh, 1), W2, b2.reshape(h, 1), w3p, b3p)

    return out[:out_c, 0]
